# initial kernel scaffold (unmeasured)
import jax
import jax.numpy as jnp
from jax import lax
from jax.experimental import pallas as pl
from jax.experimental.pallas import tpu as pltpu


def kernel(
    x,
):
    def body(*refs):
        pass

    out_shape = jax.ShapeDtypeStruct(..., jnp.float32)
    return pl.pallas_call(body, out_shape=out_shape)(...)


# baseline (device time: 444067 ns/iter reference)
import jax
import jax.numpy as jnp
from jax import lax
from jax.experimental import pallas as pl
from jax.experimental.pallas import tpu as pltpu

N_DEV = 32


def kernel(x):
    m_per, n = x.shape

    def body(x_ref, out_ref, send_sems, recv_sems):
        my_pos = lax.axis_index("i")
        left = lax.rem(my_pos + N_DEV - 1, N_DEV)
        right = lax.rem(my_pos + 1, N_DEV)

        barrier_sem = pltpu.get_barrier_semaphore()
        for nbr in (left, right):
            pl.semaphore_signal(
                barrier_sem, inc=1,
                device_id=(nbr,), device_id_type=pl.DeviceIdType.MESH,
            )
        pl.semaphore_wait(barrier_sem, 2)

        out_ref[my_pos] = x_ref[...].astype(out_ref.dtype)

        for h in range(N_DEV - 1):
            o = lax.rem(my_pos - h + N_DEV, N_DEV)
            rdma = pltpu.make_async_remote_copy(
                src_ref=out_ref.at[o],
                dst_ref=out_ref.at[o],
                send_sem=send_sems.at[h],
                recv_sem=recv_sems.at[h],
                device_id=(right,),
                device_id_type=pl.DeviceIdType.MESH,
            )
            rdma.start()
            rdma.wait()

    out = pl.pallas_call(
        body,
        out_shape=jax.ShapeDtypeStruct((N_DEV, m_per, n), jnp.bfloat16),
        in_specs=[pl.BlockSpec(memory_space=pltpu.VMEM)],
        out_specs=pl.BlockSpec(memory_space=pltpu.VMEM),
        scratch_shapes=[
            pltpu.SemaphoreType.DMA((N_DEV - 1,)),
            pltpu.SemaphoreType.DMA((N_DEV - 1,)),
        ],
        compiler_params=pltpu.CompilerParams(collective_id=0),
    )(x)
    return out.reshape(N_DEV * m_per, n)


# device time: 427628 ns/iter; 1.0384x vs baseline; 1.0384x over previous
import jax
import jax.numpy as jnp
from jax import lax
from jax.experimental import pallas as pl
from jax.experimental.pallas import tpu as pltpu

N_DEV = 32
H_R = N_DEV // 2
H_L = N_DEV - 1 - H_R


def kernel(x):
    m_per, n = x.shape

    def body(x_ref, out_ref, send_r, recv_r, send_l, recv_l):
        my_pos = lax.axis_index("i")
        left = lax.rem(my_pos + N_DEV - 1, N_DEV)
        right = lax.rem(my_pos + 1, N_DEV)

        barrier_sem = pltpu.get_barrier_semaphore()
        for nbr in (left, right):
            pl.semaphore_signal(
                barrier_sem, inc=1,
                device_id=(nbr,), device_id_type=pl.DeviceIdType.MESH,
            )
        pl.semaphore_wait(barrier_sem, 2)

        out_ref[my_pos] = x_ref[...].astype(out_ref.dtype)

        def rdma_hop(h, rightward):
            if rightward:
                o = lax.rem(my_pos - h + N_DEV, N_DEV)
                tgt, ss, rs = right, send_r, recv_r
            else:
                o = lax.rem(my_pos + h, N_DEV)
                tgt, ss, rs = left, send_l, recv_l
            return pltpu.make_async_remote_copy(
                src_ref=out_ref.at[o],
                dst_ref=out_ref.at[o],
                send_sem=ss.at[h],
                recv_sem=rs.at[h],
                device_id=(tgt,),
                device_id_type=pl.DeviceIdType.MESH,
            )

        for h in range(H_R):
            r = rdma_hop(h, True)
            r.start()
            if h < H_L:
                l = rdma_hop(h, False)
                l.start()
            r.wait_recv()
            if h < H_L:
                l.wait_recv()

        for h in range(H_R):
            rdma_hop(h, True).wait_send()
        for h in range(H_L):
            rdma_hop(h, False).wait_send()

    out = pl.pallas_call(
        body,
        out_shape=jax.ShapeDtypeStruct((N_DEV, m_per, n), jnp.bfloat16),
        in_specs=[pl.BlockSpec(memory_space=pltpu.VMEM)],
        out_specs=pl.BlockSpec(memory_space=pltpu.VMEM),
        scratch_shapes=[
            pltpu.SemaphoreType.DMA((H_R,)),
            pltpu.SemaphoreType.DMA((H_R,)),
            pltpu.SemaphoreType.DMA((H_L,)),
            pltpu.SemaphoreType.DMA((H_L,)),
        ],
        compiler_params=pltpu.CompilerParams(collective_id=0),
    )(x)
    return out.reshape(N_DEV * m_per, n)


# device time: 249039 ns/iter; 1.7831x vs baseline; 1.7171x over previous
import jax
import jax.numpy as jnp
from jax import lax
from jax.experimental import pallas as pl
from jax.experimental.pallas import tpu as pltpu

N_DEV = 32
H_R = N_DEV // 2
H_L = N_DEV - 1 - H_R


def _hamiltonian_cycle():
    path_yz = []
    for z in range(4):
        ys = range(4) if z % 2 == 0 else range(3, -1, -1)
        path_yz.extend((y, z) for y in ys)
    coords = [(0, y, z) for (y, z) in path_yz]
    coords += [(1, y, z) for (y, z) in reversed(path_yz)]

    def logical(c):
        x, y, z = c
        return 8 * z + 2 * y + (x if y % 2 == 0 else 1 - x)

    cyc = [logical(c) for c in coords]
    assert sorted(cyc) == list(range(N_DEV))
    pos = [0] * N_DEV
    for p, l in enumerate(cyc):
        pos[l] = p
    return cyc, pos


_CYC, _POS = _hamiltonian_cycle()


def kernel(x):
    m_per, n = x.shape

    def body(cyc_ref, pos_ref, x_ref, out_ref, send_r, recv_r, send_l, recv_l):
        my_pos = lax.axis_index("i")
        p = pos_ref[my_pos]
        right = cyc_ref[lax.rem(p + 1, N_DEV)]
        left = cyc_ref[lax.rem(p + N_DEV - 1, N_DEV)]

        barrier_sem = pltpu.get_barrier_semaphore()
        for nbr in (left, right):
            pl.semaphore_signal(
                barrier_sem, inc=1,
                device_id=(nbr,), device_id_type=pl.DeviceIdType.MESH,
            )
        pl.semaphore_wait(barrier_sem, 2)

        out_ref[my_pos] = x_ref[...].astype(out_ref.dtype)

        def rdma_hop(h, rightward):
            if rightward:
                o = cyc_ref[lax.rem(p - h + N_DEV, N_DEV)]
                tgt, ss, rs = right, send_r, recv_r
            else:
                o = cyc_ref[lax.rem(p + h, N_DEV)]
                tgt, ss, rs = left, send_l, recv_l
            return pltpu.make_async_remote_copy(
                src_ref=out_ref.at[o],
                dst_ref=out_ref.at[o],
                send_sem=ss.at[h],
                recv_sem=rs.at[h],
                device_id=(tgt,),
                device_id_type=pl.DeviceIdType.MESH,
            )

        for h in range(H_R):
            r = rdma_hop(h, True)
            r.start()
            if h < H_L:
                l = rdma_hop(h, False)
                l.start()
            r.wait_recv()
            if h < H_L:
                l.wait_recv()

        for h in range(H_R):
            rdma_hop(h, True).wait_send()
        for h in range(H_L):
            rdma_hop(h, False).wait_send()

    out = pl.pallas_call(
        body,
        out_shape=jax.ShapeDtypeStruct((N_DEV, m_per, n), jnp.bfloat16),
        in_specs=[
            pl.BlockSpec(memory_space=pltpu.SMEM),
            pl.BlockSpec(memory_space=pltpu.SMEM),
            pl.BlockSpec(memory_space=pltpu.VMEM),
        ],
        out_specs=pl.BlockSpec(memory_space=pltpu.VMEM),
        scratch_shapes=[
            pltpu.SemaphoreType.DMA((H_R,)),
            pltpu.SemaphoreType.DMA((H_R,)),
            pltpu.SemaphoreType.DMA((H_L,)),
            pltpu.SemaphoreType.DMA((H_L,)),
        ],
        compiler_params=pltpu.CompilerParams(collective_id=0),
    )(
        jnp.asarray(_CYC, dtype=jnp.int32),
        jnp.asarray(_POS, dtype=jnp.int32),
        x,
    )
    return out.reshape(N_DEV * m_per, n)


# device time: 221833 ns/iter; 2.0018x vs baseline; 1.1226x over previous
import jax
import jax.numpy as jnp
from jax import lax
from jax.experimental import pallas as pl
from jax.experimental.pallas import tpu as pltpu

N_DEV = 32
H_R = N_DEV // 2
H_L = N_DEV - 1 - H_R


def _hamiltonian_cycle():
    path_yz = []
    for z in range(4):
        ys = range(4) if z % 2 == 0 else range(3, -1, -1)
        path_yz.extend((y, z) for y in ys)
    coords = [(0, y, z) for (y, z) in path_yz]
    coords += [(1, y, z) for (y, z) in reversed(path_yz)]

    def logical(c):
        x, y, z = c
        return 8 * z + 2 * y + (x if y % 2 == 0 else 1 - x)

    cyc = [logical(c) for c in coords]
    assert sorted(cyc) == list(range(N_DEV))
    pos = [0] * N_DEV
    for p, l in enumerate(cyc):
        pos[l] = p
    return cyc, pos


_CYC, _POS = _hamiltonian_cycle()


def kernel(x):
    m_per, n = x.shape

    def body(cyc_ref, pos_ref, x_ref, out_ref, send_r, recv_r, send_l, recv_l):
        my_pos = lax.axis_index("i")
        p = pos_ref[my_pos]
        right = cyc_ref[lax.rem(p + 1, N_DEV)]
        left = cyc_ref[lax.rem(p + N_DEV - 1, N_DEV)]

        barrier_sem = pltpu.get_barrier_semaphore()
        for nbr in (left, right):
            pl.semaphore_signal(
                barrier_sem, inc=1,
                device_id=(nbr,), device_id_type=pl.DeviceIdType.MESH,
            )
        pl.semaphore_wait(barrier_sem, 2)

        out_ref[my_pos] = x_ref[...].astype(out_ref.dtype)

        m_sub = m_per // 2

        def rdma_hop(h, rightward, s):
            if rightward:
                o = cyc_ref[lax.rem(p - h + N_DEV, N_DEV)]
                tgt, ss, rs = right, send_r, recv_r
            else:
                o = cyc_ref[lax.rem(p + h, N_DEV)]
                tgt, ss, rs = left, send_l, recv_l
            sub = out_ref.at[o, pl.ds(s * m_sub, m_sub), :]
            return pltpu.make_async_remote_copy(
                src_ref=sub,
                dst_ref=sub,
                send_sem=ss.at[h, s],
                recv_sem=rs.at[h, s],
                device_id=(tgt,),
                device_id_type=pl.DeviceIdType.MESH,
            )

        rdma_hop(0, True, 0).start()
        rdma_hop(0, False, 0).start()
        rdma_hop(0, True, 1).start()
        rdma_hop(0, False, 1).start()
        for h in range(H_R):
            for s in (0, 1):
                rdma_hop(h, True, s).wait_recv()
                if h + 1 < H_R:
                    rdma_hop(h + 1, True, s).start()
                if h < H_L:
                    rdma_hop(h, False, s).wait_recv()
                    if h + 1 < H_L:
                        rdma_hop(h + 1, False, s).start()

        for h in range(H_R):
            for s in (0, 1):
                rdma_hop(h, True, s).wait_send()
        for h in range(H_L):
            for s in (0, 1):
                rdma_hop(h, False, s).wait_send()

    out = pl.pallas_call(
        body,
        out_shape=jax.ShapeDtypeStruct((N_DEV, m_per, n), jnp.bfloat16),
        in_specs=[
            pl.BlockSpec(memory_space=pltpu.SMEM),
            pl.BlockSpec(memory_space=pltpu.SMEM),
            pl.BlockSpec(memory_space=pltpu.VMEM),
        ],
        out_specs=pl.BlockSpec(memory_space=pltpu.VMEM),
        scratch_shapes=[
            pltpu.SemaphoreType.DMA((H_R, 2)),
            pltpu.SemaphoreType.DMA((H_R, 2)),
            pltpu.SemaphoreType.DMA((H_L, 2)),
            pltpu.SemaphoreType.DMA((H_L, 2)),
        ],
        compiler_params=pltpu.CompilerParams(collective_id=0),
    )(
        jnp.asarray(_CYC, dtype=jnp.int32),
        jnp.asarray(_POS, dtype=jnp.int32),
        x,
    )
    return out.reshape(N_DEV * m_per, n)
